# baseline (speedup 1.0000x reference)
import functools

import jax
import jax.numpy as jnp
from jax import lax
from jax.experimental import pallas as pl
from jax.experimental.pallas import tpu as pltpu

N = 10000
E = 160000
D = 128
NH2 = 128
NH = 64
M = 2500

NPAD = 12800
EP1 = 163840
EPAD = 3 * EP1
PAD_NODE = NPAD - 1

F32 = jnp.float32


def _mm_body(x_ref, w_ref, o_ref):
    o_ref[...] = jnp.dot(x_ref[...], w_ref[...], preferred_element_type=F32)


def _matmul(x, w, block_rows):
    rows = x.shape[0]
    assert rows % block_rows == 0
    k = x.shape[1]
    n = w.shape[1]
    return pl.pallas_call(
        _mm_body,
        grid=(rows // block_rows,),
        in_specs=[
            pl.BlockSpec((block_rows, k), lambda i: (i, 0)),
            pl.BlockSpec((k, n), lambda i: (0, 0)),
        ],
        out_specs=pl.BlockSpec((block_rows, n), lambda i: (i, 0)),
        out_shape=jax.ShapeDtypeStruct((rows, n), F32),
    )(x, w)


def _mid_body(agga_ref, aggb_ref, b_ref, w2_ref, a2_ref, g_ref, as_ref):
    sa = agga_ref[0] + agga_ref[1]
    sb = aggb_ref[0] + aggb_ref[1]
    z = sa[:, ZC]
    s = jnp.concatenate([sa[:, :ZC], sb], axis=1)
    h_in = s / (z + 1e-16)[:, None] + b_ref[...]
    g = jnp.dot(h_in, w2_ref[...], preferred_element_type=F32)
    g_ref[...] = g
    as_ref[...] = jnp.dot(g, a2_ref[...], preferred_element_type=F32)


def _mid_layer(agga, aggb, b1, W2, a2pad, block_rows=1280):
    grid = NPAD // block_rows
    return pl.pallas_call(
        _mid_body,
        grid=(grid,),
        in_specs=[
            pl.BlockSpec((2, block_rows, NH), lambda i: (0, i, 0)),
            pl.BlockSpec((2, block_rows, 80), lambda i: (0, i, 0)),
            pl.BlockSpec((1, NH2), lambda i: (0, 0)),
            pl.BlockSpec((NH2, NH), lambda i: (0, 0)),
            pl.BlockSpec((NH, 128), lambda i: (0, 0)),
        ],
        out_specs=[
            pl.BlockSpec((block_rows, NH), lambda i: (i, 0)),
            pl.BlockSpec((block_rows, 128), lambda i: (i, 0)),
        ],
        out_shape=[
            jax.ShapeDtypeStruct((NPAD, NH), F32),
            jax.ShapeDtypeStruct((NPAD, 128), F32),
        ],
    )(agga, aggb, b1.reshape(1, NH2), W2, a2pad)


def _h2_body(agg_ref, b_ref, o_ref):
    s = agg_ref[0] + agg_ref[1]
    z = s[:, NH]
    o_ref[...] = s[:, :NH] / (z + 1e-16)[:, None] + b_ref[...]


def _h2_layer(agg, b2, block_rows=1280):
    grid = NPAD // block_rows
    return pl.pallas_call(
        _h2_body,
        grid=(grid,),
        in_specs=[
            pl.BlockSpec((2, block_rows, 80), lambda i: (0, i, 0)),
            pl.BlockSpec((1, NH), lambda i: (0, 0)),
        ],
        out_specs=pl.BlockSpec((block_rows, NH), lambda i: (i, 0)),
        out_shape=jax.ShapeDtypeStruct((NPAD, NH), F32),
    )(agg, b2.reshape(1, NH))


def _recf_body(h_ref, w_ref, b_ref, o_ref):
    o_ref[...] = jax.nn.sigmoid(
        jnp.dot(h_ref[...], w_ref[...], preferred_element_type=F32) + b_ref[...])


def _rec_feature(h2, Wl, bl, block_rows=1024):
    grid = (12500 + block_rows - 1) // block_rows
    return pl.pallas_call(
        _recf_body,
        grid=(grid,),
        in_specs=[
            pl.BlockSpec((block_rows, NH), lambda i: (i, 0)),
            pl.BlockSpec((NH, D), lambda i: (0, 0)),
            pl.BlockSpec((1, D), lambda i: (0, 0)),
        ],
        out_specs=pl.BlockSpec((block_rows, D), lambda i: (i, 0)),
        out_shape=jax.ShapeDtypeStruct((12500, D), F32),
    )(h2, Wl, bl.reshape(1, D))


def _adj_body(a_ref, b_ref, o_ref):
    prod = lax.dot_general(a_ref[...], b_ref[...],
                           (((1,), (1,)), ((), ())),
                           preferred_element_type=F32)
    o_ref[...] = jax.nn.sigmoid(prod)


def _rec_adj(h2, bm=512, bn=4096):
    n = 12500
    gm = (n + bm - 1) // bm
    gn = (n + bn - 1) // bn
    return pl.pallas_call(
        _adj_body,
        grid=(gm, gn),
        in_specs=[
            pl.BlockSpec((bm, NH), lambda i, j: (i, 0)),
            pl.BlockSpec((bn, NH), lambda i, j: (j, 0)),
        ],
        out_specs=pl.BlockSpec((bm, bn), lambda i, j: (i, j)),
        out_shape=jax.ShapeDtypeStruct((n, n), F32),
    )(h2, h2)


from jax.experimental.pallas import tpu_sc as plsc

ZC = 48
NWORK = 32
TCH = EPAD // (NWORK * 128)
STRIPE = NPAD // 16
MPAD = 3072


def _rewire_body(src_hbm, dst_hbm, nm_hbm, sd_hbm,
                 sv_v, dv_v, nm_v, rs_v, rd_v):
    c = lax.axis_index("c")
    s = lax.axis_index("s")
    g = c * 16 + s
    pltpu.sync_copy(src_hbm.at[g], sv_v)
    pltpu.sync_copy(dst_hbm.at[g], dv_v)
    pltpu.sync_copy(nm_hbm, nm_v)

    def rw(i, carry):
        sl = pl.ds(16 * i, 16)
        s16 = sv_v[sl]
        d16 = dv_v[sl]
        nms = plsc.load_gather(nm_v, [s16])
        nmd = plsc.load_gather(nm_v, [d16])
        rs = jnp.where(nms >= 0, N + nms, s16)
        rd = jnp.where(nmd >= 0, N + nmd, d16)
        sv_v[sl] = s16 | (d16 << 16)
        rs_v[sl] = rs | (rd << 16)
        rd_v[sl] = rd | (rs << 16)
        return carry

    lax.fori_loop(0, EP1 // (32 * 16), rw, 0)

    base = 5120 * g
    pltpu.sync_copy(sv_v, sd_hbm.at[pl.ds(base, 5120)])
    pltpu.sync_copy(rs_v, sd_hbm.at[pl.ds(EP1 + base, 5120)])
    pltpu.sync_copy(rd_v, sd_hbm.at[pl.ds(2 * EP1 + base, 5120)])


def _rewire(srcp, dstp, nm_pad):
    mesh = plsc.VectorSubcoreMesh(core_axis_name="c", subcore_axis_name="s")
    f = pl.kernel(
        _rewire_body,
        out_type=jax.ShapeDtypeStruct((EPAD,), jnp.int32),
        mesh=mesh,
        compiler_params=pltpu.CompilerParams(
            needs_layout_passes=False, use_tc_tiling_on_sc=False),
        scratch_types=[
            pltpu.VMEM((5120,), jnp.int32),
            pltpu.VMEM((5120,), jnp.int32),
            pltpu.VMEM((NPAD,), jnp.int32),
            pltpu.VMEM((5120,), jnp.int32),
            pltpu.VMEM((5120,), jnp.int32),
        ],
    )
    return f(srcp, dstp, nm_pad)


def _motif_body(y_hbm, ml_hbm, out_hbm, ml_v, rows_v, out_v, sem):
    c = lax.axis_index("c")
    s = lax.axis_index("s")
    g = c * 16 + s
    pltpu.sync_copy(ml_hbm.at[g], ml_v)

    def chunk(t, carry):
        pltpu.async_copy(y_hbm.at[ml_v.at[t]], rows_v, sem).wait()
        for m in range(32):
            for k in range(8):
                sl = pl.ds(16 * k, 16)
                acc = (rows_v.at[4 * m][sl] + rows_v.at[4 * m + 1][sl]
                       + rows_v.at[4 * m + 2][sl] + rows_v.at[4 * m + 3][sl])
                out_v.at[m][sl] = acc * 0.25
        pltpu.sync_copy(out_v, out_hbm.at[pl.ds(g * 96 + t * 32, 32)])
        return carry

    lax.fori_loop(0, 3, chunk, 0)


def _motif_means(Y, ml):
    mesh = plsc.VectorSubcoreMesh(core_axis_name="c", subcore_axis_name="s")
    f = pl.kernel(
        _motif_body,
        out_type=jax.ShapeDtypeStruct((MPAD, NH2), F32),
        mesh=mesh,
        compiler_params=pltpu.CompilerParams(
            needs_layout_passes=False, use_tc_tiling_on_sc=False),
        scratch_types=[
            pltpu.VMEM((3, 128), jnp.int32),
            pltpu.VMEM((128, NH2), F32),
            pltpu.VMEM((32, NH2), F32),
            pltpu.SemaphoreType.DMA,
        ],
    )
    return f(Y, ml)


def _edge_body(width, h_hbm, as_hbm, ad_hbm, sd_hbm,
               agg_hbm,
               sd_v, sidx_v, didx_v, as_v, ad_v, rows_v, w_v,
               out_sh, sem, sem_s):
    c = lax.axis_index("c")
    s = lax.axis_index("s")
    g = c * 16 + s
    nk = width // 16

    pltpu.sync_copy(sd_hbm.at[g], sd_v)
    pltpu.sync_copy(as_hbm, as_v)
    pltpu.sync_copy(ad_hbm, ad_v)

    zero16 = jnp.zeros((16,), F32)

    def zrow(r, carry):
        for k in range(nk):
            rows_v.at[r][pl.ds(16 * k, 16)] = zero16
        return carry

    lax.fori_loop(0, 128, zrow, 0)

    for i in range(STRIPE // 128):
        pltpu.sync_copy(rows_v.at[pl.ds(0, 128)],
                        out_sh.at[pl.ds(s * STRIPE + i * 128, 128)])
    rem = STRIPE % 128
    if rem:
        pltpu.sync_copy(rows_v.at[pl.ds(0, rem)],
                        out_sh.at[pl.ds(s * STRIPE + STRIPE - rem, rem)])
    plsc.subcore_barrier()

    def step(t, carry):
        @pl.when(t >= 2)
        def _drain():
            j2 = t - 2
            buf2 = rows_v.at[pl.ds((j2 % 2) * 128, 128)]
            pltpu.make_async_copy(buf2, out_sh.at[didx_v.at[j2 % 2]],
                                  sem_s.at[j2 % 2]).wait()

        @pl.when(t < TCH)
        def _start():
            par = t % 2
            off = par * 128
            for k in range(8):
                sl = pl.ds(16 * k, 16)
                sd16 = sd_v.at[t][sl]
                si = sd16 & 0xFFFF
                di = lax.shift_right_logical(sd16, 16)
                sidx_v.at[par][sl] = si
                didx_v.at[par][sl] = di
                e = (plsc.load_gather(as_v, [si])
                     + plsc.load_gather(ad_v, [di]))
                e = jnp.maximum(e, 0.2 * e)
                w_v[pl.ds(off + 16 * k, 16)] = jnp.exp(e)
            pltpu.async_copy(h_hbm.at[sidx_v.at[par]],
                             rows_v.at[pl.ds(off, 128)], sem.at[par])

        @pl.when((t > 0) & (t <= TCH))
        def _process():
            j = t - 1
            par = j % 2
            off = par * 128
            buf = rows_v.at[pl.ds(off, 128)]
            pltpu.make_async_copy(h_hbm.at[sidx_v.at[par]], buf,
                                  sem.at[par]).wait()

            def scale(r, carry2):
                wb = plsc.load_gather(
                    w_v, [jnp.full((16,), off + r, jnp.int32)])
                row = rows_v.at[off + r]
                for k in range(nk):
                    sl = pl.ds(16 * k, 16)
                    row[sl] = row[sl] * wb
                return carry2

            lax.fori_loop(0, 128, scale, 0, unroll=4)
            pltpu.async_copy(buf, out_sh.at[didx_v.at[par]],
                             sem_s.at[par], add=True)

        return carry

    lax.fori_loop(0, TCH + 2, step, 0)
    plsc.subcore_barrier()

    pltpu.sync_copy(out_sh.at[pl.ds(s * STRIPE, STRIPE)],
                    agg_hbm.at[c].at[pl.ds(s * STRIPE, STRIPE)])


def _edge_phase(Htab, as_t, ad_t, sd_r):
    width = Htab.shape[1]
    mesh = plsc.VectorSubcoreMesh(core_axis_name="c", subcore_axis_name="s")
    f = pl.kernel(
        functools.partial(_edge_body, width),
        out_type=jax.ShapeDtypeStruct((2, NPAD, width), F32),
        mesh=mesh,
        compiler_params=pltpu.CompilerParams(
            needs_layout_passes=False, use_tc_tiling_on_sc=False),
        scratch_types=[
            pltpu.VMEM((TCH, 128), jnp.int32),
            pltpu.VMEM((2, 128), jnp.int32),
            pltpu.VMEM((2, 128), jnp.int32),
            pltpu.VMEM((NPAD,), F32),
            pltpu.VMEM((NPAD,), F32),
            pltpu.VMEM((256, width), F32),
            pltpu.VMEM((256,), F32),
            pltpu.VMEM_SHARED((NPAD, width), F32),
            pltpu.SemaphoreType.DMA((2,)),
            pltpu.SemaphoreType.DMA((2,)),
        ],
    )
    return f(Htab, as_t, ad_t, sd_r)


def kernel(x, edge_index, motif_list, W1, a1_src, a1_dst, b1,
           W2, a2_src, a2_dst, b2, Wl, bl):
    node_motif = jnp.full((N,), -1, dtype=jnp.int32)
    node_motif = node_motif.at[motif_list.reshape(-1)].set(
        jnp.repeat(jnp.arange(M, dtype=jnp.int32), 4))
    nm_pad = jnp.pad(node_motif, (0, NPAD - N), constant_values=-1)
    ei = jnp.pad(edge_index.astype(jnp.int32), ((0, 0), (0, EP1 - E)),
                 constant_values=PAD_NODE).reshape(2, NWORK, EP1 // NWORK)
    sd_r = _rewire(ei[0], ei[1], nm_pad).reshape(NWORK, TCH, 128)

    Y = _matmul(x, W1, 1000)
    mlp = jnp.zeros((MPAD, 4), jnp.int32)
    mlp = mlp.at[:M].set(motif_list.astype(jnp.int32))
    motif_rows = _motif_means(Y, mlp.reshape(NWORK, 3, 128))
    H1 = jnp.concatenate(
        [Y, motif_rows[:M], jnp.zeros((NPAD - N - M, NH2), F32)], axis=0)
    v1 = jnp.zeros((NH2, 128), F32).at[:, 0].set(a1_src).at[:, 1].set(a1_dst)
    AS1 = _matmul(H1, v1, 1600)

    as1, ad1 = AS1[:, 0], AS1[:, 1]
    zcol = jnp.zeros((NPAD, 16), F32).at[:, 0].set(1.0)
    T1a = jnp.concatenate([H1[:, :ZC], zcol], axis=1)
    agg1a = _edge_phase(T1a, as1, ad1, sd_r)
    agg1b = _edge_phase(H1[:, ZC:], as1, ad1, sd_r)

    v2 = jnp.zeros((NH, 128), F32).at[:, 0].set(a2_src).at[:, 1].set(a2_dst)
    G, AS2 = _mid_layer(agg1a, agg1b, b1, W2, v2)

    T2 = jnp.concatenate([G, zcol], axis=1)
    agg2 = _edge_phase(T2, AS2[:, 0], AS2[:, 1], sd_r)

    h2 = _h2_layer(agg2, b2)

    h2v = h2[:12500]
    rec_feature = _rec_feature(h2v, Wl, bl)
    rec_adj = _rec_adj(h2v)
    return (rec_feature, rec_adj)


# speedup vs baseline: 1.0168x; 1.0168x over previous
import functools

import jax
import jax.numpy as jnp
from jax import lax
from jax.experimental import pallas as pl
from jax.experimental.pallas import tpu as pltpu

N = 10000
E = 160000
D = 128
NH2 = 128
NH = 64
M = 2500

NPAD = 12800
EP1 = 163840
EPAD = 3 * EP1
PAD_NODE = NPAD - 1

F32 = jnp.float32


def _mm_body(x_ref, w_ref, o_ref):
    o_ref[...] = jnp.dot(x_ref[...], w_ref[...], preferred_element_type=F32)


def _matmul(x, w, block_rows):
    rows = x.shape[0]
    assert rows % block_rows == 0
    k = x.shape[1]
    n = w.shape[1]
    return pl.pallas_call(
        _mm_body,
        grid=(rows // block_rows,),
        in_specs=[
            pl.BlockSpec((block_rows, k), lambda i: (i, 0)),
            pl.BlockSpec((k, n), lambda i: (0, 0)),
        ],
        out_specs=pl.BlockSpec((block_rows, n), lambda i: (i, 0)),
        out_shape=jax.ShapeDtypeStruct((rows, n), F32),
    )(x, w)


def _mid_body(agga_ref, aggb_ref, b_ref, w2_ref, a2_ref, g_ref, as_ref):
    sa = agga_ref[0] + agga_ref[1]
    sb = aggb_ref[0] + aggb_ref[1]
    z = sa[:, ZC]
    s = jnp.concatenate([sa[:, :ZC], sb], axis=1)
    h_in = s / (z + 1e-16)[:, None] + b_ref[...]
    g = jnp.dot(h_in, w2_ref[...], preferred_element_type=F32)
    g_ref[...] = g
    as_ref[...] = jnp.dot(g, a2_ref[...], preferred_element_type=F32)


def _mid_layer(agga, aggb, b1, W2, a2pad, block_rows=1280):
    grid = NPAD // block_rows
    return pl.pallas_call(
        _mid_body,
        grid=(grid,),
        in_specs=[
            pl.BlockSpec((2, block_rows, NH), lambda i: (0, i, 0)),
            pl.BlockSpec((2, block_rows, 80), lambda i: (0, i, 0)),
            pl.BlockSpec((1, NH2), lambda i: (0, 0)),
            pl.BlockSpec((NH2, NH), lambda i: (0, 0)),
            pl.BlockSpec((NH, 128), lambda i: (0, 0)),
        ],
        out_specs=[
            pl.BlockSpec((block_rows, NH), lambda i: (i, 0)),
            pl.BlockSpec((block_rows, 128), lambda i: (i, 0)),
        ],
        out_shape=[
            jax.ShapeDtypeStruct((NPAD, NH), F32),
            jax.ShapeDtypeStruct((NPAD, 128), F32),
        ],
    )(agga, aggb, b1.reshape(1, NH2), W2, a2pad)


def _h2_body(agg_ref, b_ref, o_ref):
    s = agg_ref[0] + agg_ref[1]
    z = s[:, NH]
    o_ref[...] = s[:, :NH] / (z + 1e-16)[:, None] + b_ref[...]


def _h2_layer(agg, b2, block_rows=1280):
    grid = NPAD // block_rows
    return pl.pallas_call(
        _h2_body,
        grid=(grid,),
        in_specs=[
            pl.BlockSpec((2, block_rows, 80), lambda i: (0, i, 0)),
            pl.BlockSpec((1, NH), lambda i: (0, 0)),
        ],
        out_specs=pl.BlockSpec((block_rows, NH), lambda i: (i, 0)),
        out_shape=jax.ShapeDtypeStruct((NPAD, NH), F32),
    )(agg, b2.reshape(1, NH))


def _recf_body(h_ref, w_ref, b_ref, o_ref):
    o_ref[...] = jax.nn.sigmoid(
        jnp.dot(h_ref[...], w_ref[...], preferred_element_type=F32) + b_ref[...])


def _rec_feature(h2, Wl, bl, block_rows=1024):
    grid = (12500 + block_rows - 1) // block_rows
    return pl.pallas_call(
        _recf_body,
        grid=(grid,),
        in_specs=[
            pl.BlockSpec((block_rows, NH), lambda i: (i, 0)),
            pl.BlockSpec((NH, D), lambda i: (0, 0)),
            pl.BlockSpec((1, D), lambda i: (0, 0)),
        ],
        out_specs=pl.BlockSpec((block_rows, D), lambda i: (i, 0)),
        out_shape=jax.ShapeDtypeStruct((12500, D), F32),
    )(h2, Wl, bl.reshape(1, D))


def _adj_body(a_ref, b_ref, o_ref):
    prod = lax.dot_general(a_ref[...], b_ref[...],
                           (((1,), (1,)), ((), ())),
                           preferred_element_type=F32)
    o_ref[...] = jax.nn.sigmoid(prod)


def _rec_adj(h2, bm=2048, bn=2048):
    n = 12500
    gm = (n + bm - 1) // bm
    gn = (n + bn - 1) // bn
    return pl.pallas_call(
        _adj_body,
        grid=(gm, gn),
        in_specs=[
            pl.BlockSpec((bm, NH), lambda i, j: (i, 0)),
            pl.BlockSpec((bn, NH), lambda i, j: (j, 0)),
        ],
        out_specs=pl.BlockSpec((bm, bn), lambda i, j: (i, j)),
        out_shape=jax.ShapeDtypeStruct((n, n), F32),
    )(h2, h2)


from jax.experimental.pallas import tpu_sc as plsc

ZC = 48
NWORK = 32
TCH = EPAD // (NWORK * 128)
STRIPE = NPAD // 16
MPAD = 3072


def _rewire_body(src_hbm, dst_hbm, nm_hbm, sd_hbm,
                 sv_v, dv_v, nm_v, rs_v, rd_v):
    c = lax.axis_index("c")
    s = lax.axis_index("s")
    g = c * 16 + s
    pltpu.sync_copy(src_hbm.at[g], sv_v)
    pltpu.sync_copy(dst_hbm.at[g], dv_v)
    pltpu.sync_copy(nm_hbm, nm_v)

    def rw(i, carry):
        sl = pl.ds(16 * i, 16)
        s16 = sv_v[sl]
        d16 = dv_v[sl]
        nms = plsc.load_gather(nm_v, [s16])
        nmd = plsc.load_gather(nm_v, [d16])
        rs = jnp.where(nms >= 0, N + nms, s16)
        rd = jnp.where(nmd >= 0, N + nmd, d16)
        sv_v[sl] = s16 | (d16 << 16)
        rs_v[sl] = rs | (rd << 16)
        rd_v[sl] = rd | (rs << 16)
        return carry

    lax.fori_loop(0, EP1 // (32 * 16), rw, 0)

    base = 5120 * g
    pltpu.sync_copy(sv_v, sd_hbm.at[pl.ds(base, 5120)])
    pltpu.sync_copy(rs_v, sd_hbm.at[pl.ds(EP1 + base, 5120)])
    pltpu.sync_copy(rd_v, sd_hbm.at[pl.ds(2 * EP1 + base, 5120)])


def _rewire(srcp, dstp, nm_pad):
    mesh = plsc.VectorSubcoreMesh(core_axis_name="c", subcore_axis_name="s")
    f = pl.kernel(
        _rewire_body,
        out_type=jax.ShapeDtypeStruct((EPAD,), jnp.int32),
        mesh=mesh,
        compiler_params=pltpu.CompilerParams(
            needs_layout_passes=False, use_tc_tiling_on_sc=False),
        scratch_types=[
            pltpu.VMEM((5120,), jnp.int32),
            pltpu.VMEM((5120,), jnp.int32),
            pltpu.VMEM((NPAD,), jnp.int32),
            pltpu.VMEM((5120,), jnp.int32),
            pltpu.VMEM((5120,), jnp.int32),
        ],
    )
    return f(srcp, dstp, nm_pad)


def _motif_body(y_hbm, ml_hbm, out_hbm, ml_v, rows_v, out_v, sem):
    c = lax.axis_index("c")
    s = lax.axis_index("s")
    g = c * 16 + s
    pltpu.sync_copy(ml_hbm.at[g], ml_v)

    def chunk(t, carry):
        pltpu.async_copy(y_hbm.at[ml_v.at[t]], rows_v, sem).wait()
        for m in range(32):
            for k in range(8):
                sl = pl.ds(16 * k, 16)
                acc = (rows_v.at[4 * m][sl] + rows_v.at[4 * m + 1][sl]
                       + rows_v.at[4 * m + 2][sl] + rows_v.at[4 * m + 3][sl])
                out_v.at[m][sl] = acc * 0.25
        pltpu.sync_copy(out_v, out_hbm.at[pl.ds(g * 96 + t * 32, 32)])
        return carry

    lax.fori_loop(0, 3, chunk, 0)


def _motif_means(Y, ml):
    mesh = plsc.VectorSubcoreMesh(core_axis_name="c", subcore_axis_name="s")
    f = pl.kernel(
        _motif_body,
        out_type=jax.ShapeDtypeStruct((MPAD, NH2), F32),
        mesh=mesh,
        compiler_params=pltpu.CompilerParams(
            needs_layout_passes=False, use_tc_tiling_on_sc=False),
        scratch_types=[
            pltpu.VMEM((3, 128), jnp.int32),
            pltpu.VMEM((128, NH2), F32),
            pltpu.VMEM((32, NH2), F32),
            pltpu.SemaphoreType.DMA,
        ],
    )
    return f(Y, ml)


def _edge_body(width, h_hbm, as_hbm, ad_hbm, sd_hbm,
               agg_hbm,
               sd_v, sidx_v, didx_v, as_v, ad_v, rows_v, w_v,
               out_sh, sem, sem_s):
    c = lax.axis_index("c")
    s = lax.axis_index("s")
    g = c * 16 + s
    nk = width // 16

    pltpu.sync_copy(sd_hbm.at[g], sd_v)
    pltpu.sync_copy(as_hbm, as_v)
    pltpu.sync_copy(ad_hbm, ad_v)

    zero16 = jnp.zeros((16,), F32)

    def zrow(r, carry):
        for k in range(nk):
            rows_v.at[r][pl.ds(16 * k, 16)] = zero16
        return carry

    lax.fori_loop(0, 128, zrow, 0)

    for i in range(STRIPE // 128):
        pltpu.sync_copy(rows_v.at[pl.ds(0, 128)],
                        out_sh.at[pl.ds(s * STRIPE + i * 128, 128)])
    rem = STRIPE % 128
    if rem:
        pltpu.sync_copy(rows_v.at[pl.ds(0, rem)],
                        out_sh.at[pl.ds(s * STRIPE + STRIPE - rem, rem)])
    plsc.subcore_barrier()

    def step(t, carry):
        @pl.when(t >= 2)
        def _drain():
            j2 = t - 2
            buf2 = rows_v.at[pl.ds((j2 % 2) * 128, 128)]
            pltpu.make_async_copy(buf2, out_sh.at[didx_v.at[j2 % 2]],
                                  sem_s.at[j2 % 2]).wait()

        @pl.when(t < TCH)
        def _start():
            par = t % 2
            off = par * 128
            for k in range(8):
                sl = pl.ds(16 * k, 16)
                sd16 = sd_v.at[t][sl]
                si = sd16 & 0xFFFF
                di = lax.shift_right_logical(sd16, 16)
                sidx_v.at[par][sl] = si
                didx_v.at[par][sl] = di
                e = (plsc.load_gather(as_v, [si])
                     + plsc.load_gather(ad_v, [di]))
                e = jnp.maximum(e, 0.2 * e)
                w_v[pl.ds(off + 16 * k, 16)] = jnp.exp(e)
            pltpu.async_copy(h_hbm.at[sidx_v.at[par]],
                             rows_v.at[pl.ds(off, 128)], sem.at[par])

        @pl.when((t > 0) & (t <= TCH))
        def _process():
            j = t - 1
            par = j % 2
            off = par * 128
            buf = rows_v.at[pl.ds(off, 128)]
            pltpu.make_async_copy(h_hbm.at[sidx_v.at[par]], buf,
                                  sem.at[par]).wait()

            def scale(r, carry2):
                wb = plsc.load_gather(
                    w_v, [jnp.full((16,), off + r, jnp.int32)])
                row = rows_v.at[off + r]
                for k in range(nk):
                    sl = pl.ds(16 * k, 16)
                    row[sl] = row[sl] * wb
                return carry2

            lax.fori_loop(0, 128, scale, 0, unroll=4)
            pltpu.async_copy(buf, out_sh.at[didx_v.at[par]],
                             sem_s.at[par], add=True)

        return carry

    lax.fori_loop(0, TCH + 2, step, 0)
    plsc.subcore_barrier()

    pltpu.sync_copy(out_sh.at[pl.ds(s * STRIPE, STRIPE)],
                    agg_hbm.at[c].at[pl.ds(s * STRIPE, STRIPE)])


def _edge_phase(Htab, as_t, ad_t, sd_r):
    width = Htab.shape[1]
    mesh = plsc.VectorSubcoreMesh(core_axis_name="c", subcore_axis_name="s")
    f = pl.kernel(
        functools.partial(_edge_body, width),
        out_type=jax.ShapeDtypeStruct((2, NPAD, width), F32),
        mesh=mesh,
        compiler_params=pltpu.CompilerParams(
            needs_layout_passes=False, use_tc_tiling_on_sc=False),
        scratch_types=[
            pltpu.VMEM((TCH, 128), jnp.int32),
            pltpu.VMEM((2, 128), jnp.int32),
            pltpu.VMEM((2, 128), jnp.int32),
            pltpu.VMEM((NPAD,), F32),
            pltpu.VMEM((NPAD,), F32),
            pltpu.VMEM((256, width), F32),
            pltpu.VMEM((256,), F32),
            pltpu.VMEM_SHARED((NPAD, width), F32),
            pltpu.SemaphoreType.DMA((2,)),
            pltpu.SemaphoreType.DMA((2,)),
        ],
    )
    return f(Htab, as_t, ad_t, sd_r)


def kernel(x, edge_index, motif_list, W1, a1_src, a1_dst, b1,
           W2, a2_src, a2_dst, b2, Wl, bl):
    node_motif = jnp.full((N,), -1, dtype=jnp.int32)
    node_motif = node_motif.at[motif_list.reshape(-1)].set(
        jnp.repeat(jnp.arange(M, dtype=jnp.int32), 4))
    nm_pad = jnp.pad(node_motif, (0, NPAD - N), constant_values=-1)
    ei = jnp.pad(edge_index.astype(jnp.int32), ((0, 0), (0, EP1 - E)),
                 constant_values=PAD_NODE).reshape(2, NWORK, EP1 // NWORK)
    sd_r = _rewire(ei[0], ei[1], nm_pad).reshape(NWORK, TCH, 128)

    Y = _matmul(x, W1, 1000)
    mlp = jnp.zeros((MPAD, 4), jnp.int32)
    mlp = mlp.at[:M].set(motif_list.astype(jnp.int32))
    motif_rows = _motif_means(Y, mlp.reshape(NWORK, 3, 128))
    H1 = jnp.concatenate(
        [Y, motif_rows[:M], jnp.zeros((NPAD - N - M, NH2), F32)], axis=0)
    v1 = jnp.zeros((NH2, 128), F32).at[:, 0].set(a1_src).at[:, 1].set(a1_dst)
    AS1 = _matmul(H1, v1, 1600)

    as1, ad1 = AS1[:, 0], AS1[:, 1]
    zcol = jnp.zeros((NPAD, 16), F32).at[:, 0].set(1.0)
    T1a = jnp.concatenate([H1[:, :ZC], zcol], axis=1)
    agg1a = _edge_phase(T1a, as1, ad1, sd_r)
    agg1b = _edge_phase(H1[:, ZC:], as1, ad1, sd_r)

    v2 = jnp.zeros((NH, 128), F32).at[:, 0].set(a2_src).at[:, 1].set(a2_dst)
    G, AS2 = _mid_layer(agg1a, agg1b, b1, W2, v2)

    T2 = jnp.concatenate([G, zcol], axis=1)
    agg2 = _edge_phase(T2, AS2[:, 0], AS2[:, 1], sd_r)

    h2 = _h2_layer(agg2, b2)

    h2v = h2[:12500]
    rec_feature = _rec_feature(h2v, Wl, bl)
    rec_adj = _rec_adj(h2v)
    return (rec_feature, rec_adj)


# speedup vs baseline: 1.0493x; 1.0320x over previous
import functools

import jax
import jax.numpy as jnp
from jax import lax
from jax.experimental import pallas as pl
from jax.experimental.pallas import tpu as pltpu

N = 10000
E = 160000
D = 128
NH2 = 128
NH = 64
M = 2500

NPAD = 12800
EP1 = 163840
EPAD = 3 * EP1
PAD_NODE = NPAD - 1

F32 = jnp.float32


def _mm_body(x_ref, w_ref, o_ref):
    o_ref[...] = jnp.dot(x_ref[...], w_ref[...], preferred_element_type=F32)


def _matmul(x, w, block_rows):
    rows = x.shape[0]
    assert rows % block_rows == 0
    k = x.shape[1]
    n = w.shape[1]
    return pl.pallas_call(
        _mm_body,
        grid=(rows // block_rows,),
        in_specs=[
            pl.BlockSpec((block_rows, k), lambda i: (i, 0)),
            pl.BlockSpec((k, n), lambda i: (0, 0)),
        ],
        out_specs=pl.BlockSpec((block_rows, n), lambda i: (i, 0)),
        out_shape=jax.ShapeDtypeStruct((rows, n), F32),
    )(x, w)


def _mid_body(agga_ref, aggb_ref, b_ref, w2_ref, a2_ref, g_ref, as_ref):
    sa = agga_ref[0] + agga_ref[1]
    sb = aggb_ref[0] + aggb_ref[1]
    z = sa[:, ZC]
    s = jnp.concatenate([sa[:, :ZC], sb], axis=1)
    h_in = s / (z + 1e-16)[:, None] + b_ref[...]
    g = jnp.dot(h_in, w2_ref[...], preferred_element_type=F32)
    g_ref[...] = g
    as_ref[...] = jnp.dot(g, a2_ref[...], preferred_element_type=F32)


def _mid_layer(agga, aggb, b1, W2, a2pad, block_rows=1280):
    grid = NPAD // block_rows
    return pl.pallas_call(
        _mid_body,
        grid=(grid,),
        in_specs=[
            pl.BlockSpec((2, block_rows, NH), lambda i: (0, i, 0)),
            pl.BlockSpec((2, block_rows, 80), lambda i: (0, i, 0)),
            pl.BlockSpec((1, NH2), lambda i: (0, 0)),
            pl.BlockSpec((NH2, NH), lambda i: (0, 0)),
            pl.BlockSpec((NH, 128), lambda i: (0, 0)),
        ],
        out_specs=[
            pl.BlockSpec((block_rows, NH), lambda i: (i, 0)),
            pl.BlockSpec((block_rows, 128), lambda i: (i, 0)),
        ],
        out_shape=[
            jax.ShapeDtypeStruct((NPAD, NH), F32),
            jax.ShapeDtypeStruct((NPAD, 128), F32),
        ],
    )(agga, aggb, b1.reshape(1, NH2), W2, a2pad)


def _h2_body(agg_ref, b_ref, o_ref):
    s = agg_ref[0] + agg_ref[1]
    z = s[:, NH]
    o_ref[...] = s[:, :NH] / (z + 1e-16)[:, None] + b_ref[...]


def _h2_layer(agg, b2, block_rows=1280):
    grid = NPAD // block_rows
    return pl.pallas_call(
        _h2_body,
        grid=(grid,),
        in_specs=[
            pl.BlockSpec((2, block_rows, 80), lambda i: (0, i, 0)),
            pl.BlockSpec((1, NH), lambda i: (0, 0)),
        ],
        out_specs=pl.BlockSpec((block_rows, NH), lambda i: (i, 0)),
        out_shape=jax.ShapeDtypeStruct((NPAD, NH), F32),
    )(agg, b2.reshape(1, NH))


def _recf_body(h_ref, w_ref, b_ref, o_ref):
    o_ref[...] = jax.nn.sigmoid(
        jnp.dot(h_ref[...], w_ref[...], preferred_element_type=F32) + b_ref[...])


def _rec_feature(h2, Wl, bl, block_rows=1024):
    grid = (12500 + block_rows - 1) // block_rows
    return pl.pallas_call(
        _recf_body,
        grid=(grid,),
        in_specs=[
            pl.BlockSpec((block_rows, NH), lambda i: (i, 0)),
            pl.BlockSpec((NH, D), lambda i: (0, 0)),
            pl.BlockSpec((1, D), lambda i: (0, 0)),
        ],
        out_specs=pl.BlockSpec((block_rows, D), lambda i: (i, 0)),
        out_shape=jax.ShapeDtypeStruct((12500, D), F32),
    )(h2, Wl, bl.reshape(1, D))


def _adj_body(a_ref, b_ref, o_ref):
    prod = lax.dot_general(a_ref[...], b_ref[...],
                           (((1,), (1,)), ((), ())),
                           preferred_element_type=F32)
    o_ref[...] = jax.nn.sigmoid(prod)


def _rec_adj(h2, bm=2560, bn=2560):
    n = 12500
    gm = (n + bm - 1) // bm
    gn = (n + bn - 1) // bn
    return pl.pallas_call(
        _adj_body,
        grid=(gm, gn),
        in_specs=[
            pl.BlockSpec((bm, NH), lambda i, j: (i, 0)),
            pl.BlockSpec((bn, NH), lambda i, j: (j, 0)),
        ],
        out_specs=pl.BlockSpec((bm, bn), lambda i, j: (i, j)),
        out_shape=jax.ShapeDtypeStruct((n, n), F32),
    )(h2, h2)


from jax.experimental.pallas import tpu_sc as plsc

ZC = 48
NWORK = 32
TCH = EPAD // (NWORK * 128)
STRIPE = NPAD // 16
MPAD = 3072


def _rewire_body(src_hbm, dst_hbm, nm_hbm, sd_hbm,
                 sv_v, dv_v, nm_v, rs_v, rd_v):
    c = lax.axis_index("c")
    s = lax.axis_index("s")
    g = c * 16 + s
    pltpu.sync_copy(src_hbm.at[g], sv_v)
    pltpu.sync_copy(dst_hbm.at[g], dv_v)
    pltpu.sync_copy(nm_hbm, nm_v)

    def rw(i, carry):
        sl = pl.ds(16 * i, 16)
        s16 = sv_v[sl]
        d16 = dv_v[sl]
        nms = plsc.load_gather(nm_v, [s16])
        nmd = plsc.load_gather(nm_v, [d16])
        rs = jnp.where(nms >= 0, N + nms, s16)
        rd = jnp.where(nmd >= 0, N + nmd, d16)
        sv_v[sl] = s16 | (d16 << 16)
        rs_v[sl] = rs | (rd << 16)
        rd_v[sl] = rd | (rs << 16)
        return carry

    lax.fori_loop(0, EP1 // (32 * 16), rw, 0)

    base = 5120 * g
    pltpu.sync_copy(sv_v, sd_hbm.at[pl.ds(base, 5120)])
    pltpu.sync_copy(rs_v, sd_hbm.at[pl.ds(EP1 + base, 5120)])
    pltpu.sync_copy(rd_v, sd_hbm.at[pl.ds(2 * EP1 + base, 5120)])


def _rewire(srcp, dstp, nm_pad):
    mesh = plsc.VectorSubcoreMesh(core_axis_name="c", subcore_axis_name="s")
    f = pl.kernel(
        _rewire_body,
        out_type=jax.ShapeDtypeStruct((EPAD,), jnp.int32),
        mesh=mesh,
        compiler_params=pltpu.CompilerParams(
            needs_layout_passes=False, use_tc_tiling_on_sc=False),
        scratch_types=[
            pltpu.VMEM((5120,), jnp.int32),
            pltpu.VMEM((5120,), jnp.int32),
            pltpu.VMEM((NPAD,), jnp.int32),
            pltpu.VMEM((5120,), jnp.int32),
            pltpu.VMEM((5120,), jnp.int32),
        ],
    )
    return f(srcp, dstp, nm_pad)


def _motif_body(y_hbm, ml_hbm, out_hbm, ml_v, rows_v, out_v, sem):
    c = lax.axis_index("c")
    s = lax.axis_index("s")
    g = c * 16 + s
    pltpu.sync_copy(ml_hbm.at[g], ml_v)

    def chunk(t, carry):
        pltpu.async_copy(y_hbm.at[ml_v.at[t]], rows_v, sem).wait()
        for m in range(32):
            for k in range(8):
                sl = pl.ds(16 * k, 16)
                acc = (rows_v.at[4 * m][sl] + rows_v.at[4 * m + 1][sl]
                       + rows_v.at[4 * m + 2][sl] + rows_v.at[4 * m + 3][sl])
                out_v.at[m][sl] = acc * 0.25
        pltpu.sync_copy(out_v, out_hbm.at[pl.ds(g * 96 + t * 32, 32)])
        return carry

    lax.fori_loop(0, 3, chunk, 0)


def _motif_means(Y, ml):
    mesh = plsc.VectorSubcoreMesh(core_axis_name="c", subcore_axis_name="s")
    f = pl.kernel(
        _motif_body,
        out_type=jax.ShapeDtypeStruct((MPAD, NH2), F32),
        mesh=mesh,
        compiler_params=pltpu.CompilerParams(
            needs_layout_passes=False, use_tc_tiling_on_sc=False),
        scratch_types=[
            pltpu.VMEM((3, 128), jnp.int32),
            pltpu.VMEM((128, NH2), F32),
            pltpu.VMEM((32, NH2), F32),
            pltpu.SemaphoreType.DMA,
        ],
    )
    return f(Y, ml)


def _edge_body(width, h_hbm, as_hbm, ad_hbm, sd_hbm,
               agg_hbm,
               sd_v, sidx_v, didx_v, as_v, ad_v, rows_v, w_v,
               out_sh, sem, sem_s):
    c = lax.axis_index("c")
    s = lax.axis_index("s")
    g = c * 16 + s
    nk = width // 16

    pltpu.sync_copy(sd_hbm.at[g], sd_v)
    pltpu.sync_copy(as_hbm, as_v)
    pltpu.sync_copy(ad_hbm, ad_v)

    zero16 = jnp.zeros((16,), F32)

    def zrow(r, carry):
        for k in range(nk):
            rows_v.at[r][pl.ds(16 * k, 16)] = zero16
        return carry

    lax.fori_loop(0, 128, zrow, 0)

    for i in range(STRIPE // 128):
        pltpu.sync_copy(rows_v.at[pl.ds(0, 128)],
                        out_sh.at[pl.ds(s * STRIPE + i * 128, 128)])
    rem = STRIPE % 128
    if rem:
        pltpu.sync_copy(rows_v.at[pl.ds(0, rem)],
                        out_sh.at[pl.ds(s * STRIPE + STRIPE - rem, rem)])
    plsc.subcore_barrier()

    def step(t, carry):
        @pl.when(t >= 2)
        def _drain():
            j2 = t - 2
            buf2 = rows_v.at[pl.ds((j2 % 2) * 128, 128)]
            pltpu.make_async_copy(buf2, out_sh.at[didx_v.at[j2 % 2]],
                                  sem_s.at[j2 % 2]).wait()

        @pl.when(t < TCH)
        def _start():
            par = t % 2
            off = par * 128
            for k in range(8):
                sl = pl.ds(16 * k, 16)
                sd16 = sd_v.at[t][sl]
                si = sd16 & 0xFFFF
                di = lax.shift_right_logical(sd16, 16)
                sidx_v.at[par][sl] = si
                didx_v.at[par][sl] = di
                e = (plsc.load_gather(as_v, [si])
                     + plsc.load_gather(ad_v, [di]))
                e = jnp.maximum(e, 0.2 * e)
                w_v[pl.ds(off + 16 * k, 16)] = jnp.exp(e)
            pltpu.async_copy(h_hbm.at[sidx_v.at[par]],
                             rows_v.at[pl.ds(off, 128)], sem.at[par])

        @pl.when((t > 0) & (t <= TCH))
        def _process():
            j = t - 1
            par = j % 2
            off = par * 128
            buf = rows_v.at[pl.ds(off, 128)]
            pltpu.make_async_copy(h_hbm.at[sidx_v.at[par]], buf,
                                  sem.at[par]).wait()

            def scale(r, carry2):
                wb = plsc.load_gather(
                    w_v, [jnp.full((16,), off + r, jnp.int32)])
                row = rows_v.at[off + r]
                for k in range(nk):
                    sl = pl.ds(16 * k, 16)
                    row[sl] = row[sl] * wb
                return carry2

            lax.fori_loop(0, 128, scale, 0, unroll=4)
            pltpu.async_copy(buf, out_sh.at[didx_v.at[par]],
                             sem_s.at[par], add=True)

        return carry

    lax.fori_loop(0, TCH + 2, step, 0)
    plsc.subcore_barrier()

    pltpu.sync_copy(out_sh.at[pl.ds(s * STRIPE, STRIPE)],
                    agg_hbm.at[c].at[pl.ds(s * STRIPE, STRIPE)])


def _edge_phase(Htab, as_t, ad_t, sd_r):
    width = Htab.shape[1]
    mesh = plsc.VectorSubcoreMesh(core_axis_name="c", subcore_axis_name="s")
    f = pl.kernel(
        functools.partial(_edge_body, width),
        out_type=jax.ShapeDtypeStruct((2, NPAD, width), F32),
        mesh=mesh,
        compiler_params=pltpu.CompilerParams(
            needs_layout_passes=False, use_tc_tiling_on_sc=False),
        scratch_types=[
            pltpu.VMEM((TCH, 128), jnp.int32),
            pltpu.VMEM((2, 128), jnp.int32),
            pltpu.VMEM((2, 128), jnp.int32),
            pltpu.VMEM((NPAD,), F32),
            pltpu.VMEM((NPAD,), F32),
            pltpu.VMEM((256, width), F32),
            pltpu.VMEM((256,), F32),
            pltpu.VMEM_SHARED((NPAD, width), F32),
            pltpu.SemaphoreType.DMA((2,)),
            pltpu.SemaphoreType.DMA((2,)),
        ],
    )
    return f(Htab, as_t, ad_t, sd_r)


def kernel(x, edge_index, motif_list, W1, a1_src, a1_dst, b1,
           W2, a2_src, a2_dst, b2, Wl, bl):
    node_motif = jnp.full((N,), -1, dtype=jnp.int32)
    node_motif = node_motif.at[motif_list.reshape(-1)].set(
        jnp.repeat(jnp.arange(M, dtype=jnp.int32), 4))
    nm_pad = jnp.pad(node_motif, (0, NPAD - N), constant_values=-1)
    ei = jnp.pad(edge_index.astype(jnp.int32), ((0, 0), (0, EP1 - E)),
                 constant_values=PAD_NODE).reshape(2, NWORK, EP1 // NWORK)
    sd_r = _rewire(ei[0], ei[1], nm_pad).reshape(NWORK, TCH, 128)

    Y = _matmul(x, W1, 1000)
    mlp = jnp.zeros((MPAD, 4), jnp.int32)
    mlp = mlp.at[:M].set(motif_list.astype(jnp.int32))
    motif_rows = _motif_means(Y, mlp.reshape(NWORK, 3, 128))
    H1 = jnp.concatenate(
        [Y, motif_rows[:M], jnp.zeros((NPAD - N - M, NH2), F32)], axis=0)
    v1 = jnp.zeros((NH2, 128), F32).at[:, 0].set(a1_src).at[:, 1].set(a1_dst)
    AS1 = _matmul(H1, v1, 1600)

    as1, ad1 = AS1[:, 0], AS1[:, 1]
    zcol = jnp.zeros((NPAD, 16), F32).at[:, 0].set(1.0)
    T1a = jnp.concatenate([H1[:, :ZC], zcol], axis=1)
    agg1a = _edge_phase(T1a, as1, ad1, sd_r)
    agg1b = _edge_phase(H1[:, ZC:], as1, ad1, sd_r)

    v2 = jnp.zeros((NH, 128), F32).at[:, 0].set(a2_src).at[:, 1].set(a2_dst)
    G, AS2 = _mid_layer(agg1a, agg1b, b1, W2, v2)

    T2 = jnp.concatenate([G, zcol], axis=1)
    agg2 = _edge_phase(T2, AS2[:, 0], AS2[:, 1], sd_r)

    h2 = _h2_layer(agg2, b2)

    h2v = h2[:12500]
    rec_feature = _rec_feature(h2v, Wl, bl)
    rec_adj = _rec_adj(h2v)
    return (rec_feature, rec_adj)
